# Initial kernel scaffold; baseline (speedup 1.0000x reference)
import functools

import jax
import jax.numpy as jnp
from jax import lax
from jax.experimental import pallas as pl
from jax.experimental.pallas import tpu as pltpu
from jax.experimental.pallas import tpu_sc as plsc

F = 128
NB = 8
E = 30000
T = 90000
TPAD = 90112
EBLK = 512
TBLK = 512
NEB = (E + EBLK - 1) // EBLK
NTB = TPAD // TBLK
NC = 2
NS = 16
CHUNK = 128
FH = F // NC
EPS = E // NS

_f32 = jnp.float32


def _pre_body(x_ref, rbf_ref, wji_ref, bji_ref, wkj_ref, bkj_ref, wrbf_ref,
              xji_ref, xkj_ref):
    xb = x_ref[...]
    xji = jnp.dot(xb, wji_ref[...], preferred_element_type=_f32) + bji_ref[...]
    xji_ref[...] = jax.nn.silu(xji)
    xkj = jnp.dot(xb, wkj_ref[...], preferred_element_type=_f32) + bkj_ref[...]
    g = jnp.dot(rbf_ref[...], wrbf_ref[...], preferred_element_type=_f32)
    xkj_ref[...] = jax.nn.silu(xkj) * g


def _pre_call(x, rbf, W_ji, b_ji, W_kj, b_kj, W_rbf):
    row = pl.BlockSpec((EBLK, F), lambda i: (i, 0))
    full = lambda shape: pl.BlockSpec(shape, lambda i, s=shape: tuple(0 for _ in s))
    return pl.pallas_call(
        _pre_body,
        grid=(NEB,),
        in_specs=[row,
                  pl.BlockSpec((EBLK, 6), lambda i: (i, 0)),
                  full((F, F)), full((1, F)), full((F, F)), full((1, F)),
                  full((6, F))],
        out_specs=[row, row],
        out_shape=[jax.ShapeDtypeStruct((E, F), _f32),
                   jax.ShapeDtypeStruct((E, F), _f32)],
    )(x, rbf, W_ji, b_ji, W_kj, b_kj, W_rbf)


@functools.partial(
    pl.kernel,
    out_type=jax.ShapeDtypeStruct((TPAD, F), _f32),
    mesh=plsc.VectorSubcoreMesh(core_axis_name="c", subcore_axis_name="s"),
    scratch_types=[pltpu.VMEM((CHUNK,), jnp.int32),
                   pltpu.VMEM((CHUNK, F), _f32),
                   pltpu.SemaphoreType.DMA],
)
def _gather_sc(table_hbm, idx_hbm, out_hbm, idx_v, rows_v, sem):
    wid = lax.axis_index("s") * NC + lax.axis_index("c")
    per_w = TPAD // (NC * NS)
    base = wid * per_w

    def body(k, carry):
        off = base + k * CHUNK
        pltpu.sync_copy(idx_hbm.at[pl.ds(off, CHUNK)], idx_v)
        pltpu.async_copy(table_hbm.at[idx_v], rows_v, sem).wait()
        pltpu.sync_copy(rows_v, out_hbm.at[pl.ds(off, CHUNK)])
        return carry

    lax.fori_loop(0, per_w // CHUNK, body, 0)


def _bilin_body(sbf_ref, g_ref, wsbf_ref, wf_ref, m_ref):
    pid = pl.program_id(0)
    sp = jnp.dot(sbf_ref[...], wsbf_ref[...], preferred_element_type=_f32)
    rows = pid * TBLK + lax.broadcasted_iota(jnp.int32, (TBLK, 1), 0)
    sp = jnp.where(rows < T, sp, 0.0)
    gb = g_ref[...]
    acc = jnp.zeros((TBLK, F), _f32)
    for j in range(NB):
        acc += sp[:, j:j + 1] * jnp.dot(gb, wf_ref[j],
                                        preferred_element_type=_f32)
    m_ref[...] = acc


def _bilin_call(sbf, G, W_sbf, Wf):
    return pl.pallas_call(
        _bilin_body,
        grid=(NTB,),
        in_specs=[pl.BlockSpec((TBLK, 42), lambda i: (i, 0)),
                  pl.BlockSpec((TBLK, F), lambda i: (i, 0)),
                  pl.BlockSpec((42, NB), lambda i: (0, 0)),
                  pl.BlockSpec((NB, F, F), lambda i: (0, 0, 0))],
        out_specs=pl.BlockSpec((TBLK, F), lambda i: (i, 0)),
        out_shape=jax.ShapeDtypeStruct((TPAD, F), _f32),
    )(sbf, G, W_sbf, Wf)


@functools.partial(
    pl.kernel,
    out_type=jax.ShapeDtypeStruct((E, F), _f32),
    mesh=plsc.VectorSubcoreMesh(core_axis_name="c", subcore_axis_name="s"),
    scratch_types=[pltpu.VMEM((CHUNK,), jnp.int32),
                   pltpu.VMEM((CHUNK, FH), _f32),
                   pltpu.VMEM_SHARED((E, FH), _f32)],
)
def _scatter_sc(m_hbm, idx_hbm, zeros_hbm, out_hbm, idx_v, mbuf, acc):
    c = lax.axis_index("c")
    s = lax.axis_index("s")
    col0 = c * FH
    pltpu.sync_copy(zeros_hbm.at[pl.ds(s * EPS, EPS)],
                    acc.at[pl.ds(s * EPS, EPS)])
    plsc.subcore_barrier()
    per_s = TPAD // NS
    base = s * per_s

    def body(k, carry):
        off = base + k * CHUNK
        pltpu.sync_copy(idx_hbm.at[pl.ds(off, CHUNK)], idx_v)
        pltpu.sync_copy(m_hbm.at[pl.ds(off, CHUNK), pl.ds(col0, FH)], mbuf)
        pltpu.sync_copy(mbuf, acc.at[idx_v], add=True)
        return carry

    lax.fori_loop(0, per_s // CHUNK, body, 0)
    plsc.subcore_barrier()
    pltpu.sync_copy(acc.at[pl.ds(s * EPS, EPS)],
                    out_hbm.at[pl.ds(s * EPS, EPS), pl.ds(col0, FH)])


def _post_body(x_ref, xji_ref, red_ref,
               w1, b1, w2, b2, w3, b3, w4, b4, w5, b5, w6, b6, w7, b7,
               out_ref):
    act = jax.nn.silu

    def lin(v, w, b):
        return jnp.dot(v, w[...], preferred_element_type=_f32) + b[...]

    x2 = xji_ref[...] + red_ref[...]
    h = act(lin(x2, w1, b1))
    h = act(lin(h, w2, b2))
    x2 = x2 + h
    x2 = act(lin(x2, w3, b3))
    xo = x_ref[...] + x2
    h = act(lin(xo, w4, b4))
    h = act(lin(h, w5, b5))
    xo = xo + h
    h = act(lin(xo, w6, b6))
    h = act(lin(h, w7, b7))
    out_ref[...] = xo


def _post_call(x, x_ji, red, *wbs):
    row = pl.BlockSpec((EBLK, F), lambda i: (i, 0))
    wspec = pl.BlockSpec((F, F), lambda i: (0, 0))
    bspec = pl.BlockSpec((1, F), lambda i: (0, 0))
    return pl.pallas_call(
        _post_body,
        grid=(NEB,),
        in_specs=[row, row, row] + [wspec, bspec] * 7,
        out_specs=row,
        out_shape=jax.ShapeDtypeStruct((E, F), _f32),
    )(x, x_ji, red, *wbs)


def kernel(x, rbf, sbf, id_expand_kj, id_reduce_ji,
           W_rbf, W_sbf, W_ji, b_ji, W_kj, b_kj, W_bilin,
           W_bs0_0, b_bs0_0, W_bs0_1, b_bs0_1,
           W_fbs, b_fbs,
           W_as0_0, b_as0_0, W_as0_1, b_as0_1,
           W_as1_0, b_as1_0, W_as1_1, b_as1_1):
    b2 = lambda b: b.reshape(1, F)
    x_ji, x_kj = _pre_call(x, rbf, W_ji, b2(b_ji), W_kj, b2(b_kj), W_rbf)

    ide = jnp.pad(id_expand_kj.astype(jnp.int32), (0, TPAD - T))
    G = _gather_sc(x_kj, ide)

    Wf = jnp.transpose(W_bilin, (1, 2, 0))
    m = _bilin_call(sbf, G, W_sbf, Wf)

    idr = jnp.pad(id_reduce_ji.astype(jnp.int32), (0, TPAD - T))
    zeros = jnp.zeros((E, FH), _f32)
    red = _scatter_sc(m, idr, zeros)

    return _post_call(x, x_ji, red,
                      W_bs0_0, b2(b_bs0_0), W_bs0_1, b2(b_bs0_1),
                      W_fbs, b2(b_fbs),
                      W_as0_0, b2(b_as0_0), W_as0_1, b2(b_as0_1),
                      W_as1_0, b2(b_as1_0), W_as1_1, b2(b_as1_1))


# speedup vs baseline: 1.8101x; 1.8101x over previous
import functools

import jax
import jax.numpy as jnp
from jax import lax
from jax.experimental import pallas as pl
from jax.experimental.pallas import tpu as pltpu
from jax.experimental.pallas import tpu_sc as plsc

F = 128
NB = 8
E = 30000
T = 90000
TPAD = 90112
EBLK = 512
TBLK = 512
NEB = (E + EBLK - 1) // EBLK
NTB = TPAD // TBLK
NC = 2
NS = 16
CHUNK = 128
NR = 4
EPAD = 30720
ERNG = EPAD // NR
TRASH = CHUNK

_f32 = jnp.float32


def _pre_body(x_ref, rbf_ref, wji_ref, bji_ref, wkj_ref, bkj_ref, wrbf_ref,
              xji_ref, xkj_ref):
    xb = x_ref[...]
    xji = jnp.dot(xb, wji_ref[...], preferred_element_type=_f32) + bji_ref[...]
    xji_ref[...] = jax.nn.silu(xji)
    xkj = jnp.dot(xb, wkj_ref[...], preferred_element_type=_f32) + bkj_ref[...]
    g = jnp.dot(rbf_ref[...], wrbf_ref[...], preferred_element_type=_f32)
    xkj_ref[...] = jax.nn.silu(xkj) * g


def _pre_call(x, rbf, W_ji, b_ji, W_kj, b_kj, W_rbf):
    row = pl.BlockSpec((EBLK, F), lambda i: (i, 0))
    full = lambda shape: pl.BlockSpec(shape, lambda i, s=shape: tuple(0 for _ in s))
    return pl.pallas_call(
        _pre_body,
        grid=(NEB,),
        in_specs=[row,
                  pl.BlockSpec((EBLK, 6), lambda i: (i, 0)),
                  full((F, F)), full((1, F)), full((F, F)), full((1, F)),
                  full((6, F))],
        out_specs=[row, row],
        out_shape=[jax.ShapeDtypeStruct((E, F), _f32),
                   jax.ShapeDtypeStruct((E, F), _f32)],
    )(x, rbf, W_ji, b_ji, W_kj, b_kj, W_rbf)


@functools.cache
def _get_gather_sc():
    return pl.kernel(
        _gather_body,
        out_type=jax.ShapeDtypeStruct((TPAD, F), _f32),
        mesh=plsc.VectorSubcoreMesh(core_axis_name="c", subcore_axis_name="s",
                                    num_cores=NC, num_subcores=NS),
        scratch_types=[pltpu.VMEM((CHUNK,), jnp.int32),
                       pltpu.VMEM((CHUNK, F), _f32),
                       pltpu.SemaphoreType.DMA],
    )


def _gather_body(table_hbm, idx_hbm, out_hbm, idx_v, rows_v, sem):
    wid = lax.axis_index("s") * NC + lax.axis_index("c")
    per_w = TPAD // (NC * NS)
    base = wid * per_w

    def body(k, carry):
        off = base + k * CHUNK
        pltpu.sync_copy(idx_hbm.at[pl.ds(off, CHUNK)], idx_v)
        pltpu.async_copy(table_hbm.at[idx_v], rows_v, sem).wait()
        pltpu.sync_copy(rows_v, out_hbm.at[pl.ds(off, CHUNK)])
        return carry

    lax.fori_loop(0, per_w // CHUNK, body, 0)


def _bilin_body(sbf_ref, g_ref, wsbf_ref, wf_ref, m_ref):
    pid = pl.program_id(0)
    sp = jnp.dot(sbf_ref[...], wsbf_ref[...], preferred_element_type=_f32)
    rows = pid * TBLK + lax.broadcasted_iota(jnp.int32, (TBLK, 1), 0)
    sp = jnp.where(rows < T, sp, 0.0)
    gb = g_ref[...]
    acc = jnp.zeros((TBLK, F), _f32)
    for j in range(NB):
        acc += sp[:, j:j + 1] * jnp.dot(gb, wf_ref[j],
                                        preferred_element_type=_f32)
    m_ref[...] = acc


def _bilin_call(sbf, G, W_sbf, Wf):
    return pl.pallas_call(
        _bilin_body,
        grid=(NTB,),
        in_specs=[pl.BlockSpec((TBLK, 42), lambda i: (i, 0)),
                  pl.BlockSpec((TBLK, F), lambda i: (i, 0)),
                  pl.BlockSpec((42, NB), lambda i: (0, 0)),
                  pl.BlockSpec((NB, F, F), lambda i: (0, 0, 0))],
        out_specs=pl.BlockSpec((TBLK, F), lambda i: (i, 0)),
        out_shape=jax.ShapeDtypeStruct((TPAD, F), _f32),
    )(sbf, G, W_sbf, Wf)


@functools.cache
def _get_scatter_sc():
    return pl.kernel(
        _scatter_body,
        out_type=jax.ShapeDtypeStruct((EPAD, F), _f32),
        mesh=plsc.VectorSubcoreMesh(core_axis_name="c", subcore_axis_name="s",
                                    num_cores=NC, num_subcores=NS),
        scratch_types=[pltpu.VMEM((1, CHUNK), jnp.int32),
                       pltpu.VMEM((1, CHUNK), jnp.int32),
                       pltpu.VMEM((CHUNK, F), _f32),
                       pltpu.VMEM_SHARED((ERNG + TRASH, F), _f32)],
    )


def _scatter_body(m_hbm, idx_hbm, zeros_hbm, out_hbm, idx_v, idx_t, mbuf, acc):
    c = lax.axis_index("c")
    s = lax.axis_index("s")
    per_s = TPAD // NS
    base = s * per_s

    for p in range(NR // NC):
        r = NC * c + p
        e0 = r * ERNG

        @pl.when(s == 0)
        def _():
            pltpu.sync_copy(zeros_hbm, acc)
        plsc.subcore_barrier()

        def body(k, carry):
            off = base + k * CHUNK
            pltpu.sync_copy(idx_hbm.at[pl.ds(off, CHUNK)], idx_v.at[0])
            pltpu.sync_copy(m_hbm.at[pl.ds(off, CHUNK)], mbuf)
            for j in range(CHUNK // 16):
                v = idx_v[0, pl.ds(16 * j, 16)] - e0
                trash = ERNG + (16 * j) % TRASH + lax.iota(jnp.int32, 16)
                ok = (v >= 0) & (v < ERNG)
                idx_t[0, pl.ds(16 * j, 16)] = jnp.where(ok, v, trash)
            pltpu.sync_copy(mbuf, acc.at[idx_t.at[0]], add=True)
            return carry
        lax.fori_loop(0, per_s // CHUNK, body, 0)
        plsc.subcore_barrier()

        @pl.when(s == 0)
        def _():
            pltpu.sync_copy(acc.at[pl.ds(0, ERNG)], out_hbm.at[pl.ds(e0, ERNG)])


def _post_body(x_ref, xji_ref, red_ref,
               w1, b1, w2, b2, w3, b3, w4, b4, w5, b5, w6, b6, w7, b7,
               out_ref):
    act = jax.nn.silu

    def lin(v, w, b):
        return jnp.dot(v, w[...], preferred_element_type=_f32) + b[...]

    x2 = xji_ref[...] + red_ref[...]
    h = act(lin(x2, w1, b1))
    h = act(lin(h, w2, b2))
    x2 = x2 + h
    x2 = act(lin(x2, w3, b3))
    xo = x_ref[...] + x2
    h = act(lin(xo, w4, b4))
    h = act(lin(h, w5, b5))
    xo = xo + h
    h = act(lin(xo, w6, b6))
    h = act(lin(h, w7, b7))
    out_ref[...] = xo + h


def _post_call(x, x_ji, red, *wbs):
    row = pl.BlockSpec((EBLK, F), lambda i: (i, 0))
    wspec = pl.BlockSpec((F, F), lambda i: (0, 0))
    bspec = pl.BlockSpec((1, F), lambda i: (0, 0))
    return pl.pallas_call(
        _post_body,
        grid=(NEB,),
        in_specs=[row, row, row] + [wspec, bspec] * 7,
        out_specs=row,
        out_shape=jax.ShapeDtypeStruct((E, F), _f32),
    )(x, x_ji, red, *wbs)


def kernel(x, rbf, sbf, id_expand_kj, id_reduce_ji,
           W_rbf, W_sbf, W_ji, b_ji, W_kj, b_kj, W_bilin,
           W_bs0_0, b_bs0_0, W_bs0_1, b_bs0_1,
           W_fbs, b_fbs,
           W_as0_0, b_as0_0, W_as0_1, b_as0_1,
           W_as1_0, b_as1_0, W_as1_1, b_as1_1):
    b2 = lambda b: b.reshape(1, F)
    x_ji, x_kj = _pre_call(x, rbf, W_ji, b2(b_ji), W_kj, b2(b_kj), W_rbf)

    ide = jnp.pad(id_expand_kj.astype(jnp.int32), (0, TPAD - T))
    G = _get_gather_sc()(x_kj, ide)

    Wf = jnp.transpose(W_bilin, (1, 2, 0))
    m = _bilin_call(sbf, G, W_sbf, Wf)

    idr = jnp.pad(id_reduce_ji.astype(jnp.int32), (0, TPAD - T))
    zeros = jnp.zeros((ERNG + TRASH, F), _f32)
    red = _get_scatter_sc()(m, idr, zeros)

    return _post_call(x, x_ji, red,
                      W_bs0_0, b2(b_bs0_0), W_bs0_1, b2(b_bs0_1),
                      W_fbs, b2(b_fbs),
                      W_as0_0, b2(b_as0_0), W_as0_1, b2(b_as0_1),
                      W_as1_0, b2(b_as1_0), W_as1_1, b2(b_as1_1))


# speedup vs baseline: 2.0108x; 1.1109x over previous
import functools

import jax
import jax.numpy as jnp
from jax import lax
from jax.experimental import pallas as pl
from jax.experimental.pallas import tpu as pltpu
from jax.experimental.pallas import tpu_sc as plsc

F = 128
NB = 8
E = 30000
T = 90000
TPAD = 90112
EBLK = 512
TBLK = 512
NEB = (E + EBLK - 1) // EBLK
NTB = TPAD // TBLK
NC = 2
NS = 16
CHUNK = 128
NR = 4
EPAD = 30720
ERNG = EPAD // NR
TRASH = CHUNK

_f32 = jnp.float32


def _pre_body(x_ref, rbf_ref, wji_ref, bji_ref, wkj_ref, bkj_ref, wrbf_ref,
              xji_ref, xkj_ref):
    xb = x_ref[...]
    xji = jnp.dot(xb, wji_ref[...], preferred_element_type=_f32) + bji_ref[...]
    xji_ref[...] = jax.nn.silu(xji)
    xkj = jnp.dot(xb, wkj_ref[...], preferred_element_type=_f32) + bkj_ref[...]
    g = jnp.dot(rbf_ref[...], wrbf_ref[...], preferred_element_type=_f32)
    xkj_ref[...] = jax.nn.silu(xkj) * g


def _pre_call(x, rbf, W_ji, b_ji, W_kj, b_kj, W_rbf):
    row = pl.BlockSpec((EBLK, F), lambda i: (i, 0))
    full = lambda shape: pl.BlockSpec(shape, lambda i, s=shape: tuple(0 for _ in s))
    return pl.pallas_call(
        _pre_body,
        grid=(NEB,),
        in_specs=[row,
                  pl.BlockSpec((EBLK, 6), lambda i: (i, 0)),
                  full((F, F)), full((1, F)), full((F, F)), full((1, F)),
                  full((6, F))],
        out_specs=[row, row],
        out_shape=[jax.ShapeDtypeStruct((E, F), _f32),
                   jax.ShapeDtypeStruct((E, F), _f32)],
    )(x, rbf, W_ji, b_ji, W_kj, b_kj, W_rbf)


GC = 64


@functools.cache
def _get_gather_sc():
    return pl.kernel(
        _gather_body,
        out_type=jax.ShapeDtypeStruct((TPAD, F), _f32),
        mesh=plsc.VectorSubcoreMesh(core_axis_name="c", subcore_axis_name="s",
                                    num_cores=NC, num_subcores=NS),
        scratch_types=[pltpu.VMEM((1, GC), jnp.int32),
                       pltpu.VMEM((1, GC), jnp.int32),
                       pltpu.VMEM((GC, F), _f32),
                       pltpu.VMEM((GC, F), _f32),
                       pltpu.SemaphoreType.DMA,
                       pltpu.SemaphoreType.DMA,
                       pltpu.SemaphoreType.DMA,
                       pltpu.SemaphoreType.DMA],
    )


def _gather_body(table_hbm, idx_hbm, out_hbm,
                 idx_a, idx_b, rows_a, rows_b, sga, sgb, soa, sob):
    wid = lax.axis_index("s") * NC + lax.axis_index("c")
    per_w = TPAD // (NC * NS)
    base = wid * per_w

    def body(kk, carry):
        o0 = base + kk * (2 * GC)
        o1 = o0 + GC
        pltpu.sync_copy(idx_hbm.at[pl.ds(o0, GC)], idx_a.at[0])
        ha = pltpu.async_copy(table_hbm.at[idx_a.at[0]], rows_a, sga)
        pltpu.sync_copy(idx_hbm.at[pl.ds(o1, GC)], idx_b.at[0])
        hb = pltpu.async_copy(table_hbm.at[idx_b.at[0]], rows_b, sgb)
        ha.wait()
        hoa = pltpu.async_copy(rows_a, out_hbm.at[pl.ds(o0, GC)], soa)
        hb.wait()
        hob = pltpu.async_copy(rows_b, out_hbm.at[pl.ds(o1, GC)], sob)
        hoa.wait()
        hob.wait()
        return carry

    lax.fori_loop(0, per_w // (2 * GC), body, 0)


def _bilin_body(sbf_ref, g_ref, wsbf_ref, wf_ref, m_ref):
    pid = pl.program_id(0)
    sp = jnp.dot(sbf_ref[...], wsbf_ref[...], preferred_element_type=_f32)
    rows = pid * TBLK + lax.broadcasted_iota(jnp.int32, (TBLK, 1), 0)
    sp = jnp.where(rows < T, sp, 0.0)
    gb = g_ref[...]
    acc = jnp.zeros((TBLK, F), _f32)
    for j in range(NB):
        acc += sp[:, j:j + 1] * jnp.dot(gb, wf_ref[j],
                                        preferred_element_type=_f32)
    m_ref[...] = acc


def _bilin_call(sbf, G, W_sbf, Wf):
    return pl.pallas_call(
        _bilin_body,
        grid=(NTB,),
        in_specs=[pl.BlockSpec((TBLK, 42), lambda i: (i, 0)),
                  pl.BlockSpec((TBLK, F), lambda i: (i, 0)),
                  pl.BlockSpec((42, NB), lambda i: (0, 0)),
                  pl.BlockSpec((NB, F, F), lambda i: (0, 0, 0))],
        out_specs=pl.BlockSpec((TBLK, F), lambda i: (i, 0)),
        out_shape=jax.ShapeDtypeStruct((TPAD, F), _f32),
    )(sbf, G, W_sbf, Wf)


EPS = ERNG // NS
_WCH = ((0, 128), (128, 128), (256, 128), (384, 96))


@functools.cache
def _get_scatter_sc():
    return pl.kernel(
        _scatter_body,
        out_type=jax.ShapeDtypeStruct((EPAD, F), _f32),
        mesh=plsc.VectorSubcoreMesh(core_axis_name="c", subcore_axis_name="s",
                                    num_cores=NC, num_subcores=NS),
        scratch_types=[pltpu.VMEM((1, CHUNK), jnp.int32),
                       pltpu.VMEM((1, CHUNK), jnp.int32),
                       pltpu.VMEM((1, CHUNK), jnp.int32),
                       pltpu.VMEM((CHUNK, F), _f32),
                       pltpu.VMEM((CHUNK, F), _f32),
                       pltpu.VMEM_SHARED((ERNG + TRASH, F), _f32),
                       pltpu.SemaphoreType.DMA,
                       pltpu.SemaphoreType.DMA],
    )


def _scatter_body(m_hbm, idx_hbm, zeros_hbm, out_hbm,
                  idx_v, idx_ta, idx_tb, mbuf_a, mbuf_b, acc, sma, smb):
    c = lax.axis_index("c")
    s = lax.axis_index("s")
    per_s = TPAD // NS
    base = s * per_s

    def transform(idx_t, e0):
        for j in range(CHUNK // 16):
            v = idx_v[0, pl.ds(16 * j, 16)] - e0
            trash = ERNG + (16 * j) % TRASH + lax.iota(jnp.int32, 16)
            ok = (v >= 0) & (v < ERNG)
            idx_t[0, pl.ds(16 * j, 16)] = jnp.where(ok, v, trash)

    for p in range(NR // NC):
        r = NC * c + p
        e0 = r * ERNG

        pltpu.sync_copy(zeros_hbm, mbuf_a)
        for off, sz in _WCH:
            pltpu.sync_copy(mbuf_a.at[pl.ds(0, sz)],
                            acc.at[pl.ds(s * EPS + off, sz)])
        plsc.subcore_barrier()

        def body(kk, carry):
            o0 = base + kk * (2 * CHUNK)
            o1 = o0 + CHUNK
            pltpu.sync_copy(idx_hbm.at[pl.ds(o0, CHUNK)], idx_v.at[0])
            transform(idx_ta, e0)
            ha = pltpu.async_copy(m_hbm.at[pl.ds(o0, CHUNK)], mbuf_a, sma)
            pltpu.sync_copy(idx_hbm.at[pl.ds(o1, CHUNK)], idx_v.at[0])
            transform(idx_tb, e0)
            hb = pltpu.async_copy(m_hbm.at[pl.ds(o1, CHUNK)], mbuf_b, smb)
            ha.wait()
            pltpu.sync_copy(mbuf_a, acc.at[idx_ta.at[0]], add=True)
            hb.wait()
            pltpu.sync_copy(mbuf_b, acc.at[idx_tb.at[0]], add=True)
            return carry
        lax.fori_loop(0, per_s // (2 * CHUNK), body, 0)
        plsc.subcore_barrier()

        for off, sz in _WCH:
            pltpu.sync_copy(acc.at[pl.ds(s * EPS + off, sz)],
                            mbuf_a.at[pl.ds(0, sz)])
            pltpu.sync_copy(mbuf_a.at[pl.ds(0, sz)],
                            out_hbm.at[pl.ds(e0 + s * EPS + off, sz)])


def _post_body(x_ref, xji_ref, red_ref,
               w1, b1, w2, b2, w3, b3, w4, b4, w5, b5, w6, b6, w7, b7,
               out_ref):
    act = jax.nn.silu

    def lin(v, w, b):
        return jnp.dot(v, w[...], preferred_element_type=_f32) + b[...]

    x2 = xji_ref[...] + red_ref[...]
    h = act(lin(x2, w1, b1))
    h = act(lin(h, w2, b2))
    x2 = x2 + h
    x2 = act(lin(x2, w3, b3))
    xo = x_ref[...] + x2
    h = act(lin(xo, w4, b4))
    h = act(lin(h, w5, b5))
    xo = xo + h
    h = act(lin(xo, w6, b6))
    h = act(lin(h, w7, b7))
    out_ref[...] = xo + h


def _post_call(x, x_ji, red, *wbs):
    row = pl.BlockSpec((EBLK, F), lambda i: (i, 0))
    wspec = pl.BlockSpec((F, F), lambda i: (0, 0))
    bspec = pl.BlockSpec((1, F), lambda i: (0, 0))
    return pl.pallas_call(
        _post_body,
        grid=(NEB,),
        in_specs=[row, row, row] + [wspec, bspec] * 7,
        out_specs=row,
        out_shape=jax.ShapeDtypeStruct((E, F), _f32),
    )(x, x_ji, red, *wbs)


def kernel(x, rbf, sbf, id_expand_kj, id_reduce_ji,
           W_rbf, W_sbf, W_ji, b_ji, W_kj, b_kj, W_bilin,
           W_bs0_0, b_bs0_0, W_bs0_1, b_bs0_1,
           W_fbs, b_fbs,
           W_as0_0, b_as0_0, W_as0_1, b_as0_1,
           W_as1_0, b_as1_0, W_as1_1, b_as1_1):
    b2 = lambda b: b.reshape(1, F)
    x_ji, x_kj = _pre_call(x, rbf, W_ji, b2(b_ji), W_kj, b2(b_kj), W_rbf)

    ide = jnp.pad(id_expand_kj.astype(jnp.int32), (0, TPAD - T))
    G = _get_gather_sc()(x_kj, ide)

    Wf = jnp.transpose(W_bilin, (1, 2, 0))
    m = _bilin_call(sbf, G, W_sbf, Wf)

    idr = jnp.pad(id_reduce_ji.astype(jnp.int32), (0, TPAD - T))
    zeros = jnp.zeros((CHUNK, F), _f32)
    red = _get_scatter_sc()(m, idr, zeros)

    return _post_call(x, x_ji, red,
                      W_bs0_0, b2(b_bs0_0), W_bs0_1, b2(b_bs0_1),
                      W_fbs, b2(b_fbs),
                      W_as0_0, b2(b_as0_0), W_as0_1, b2(b_as0_1),
                      W_as1_0, b2(b_as1_0), W_as1_1, b2(b_as1_1))


# speedup vs baseline: 2.4117x; 1.1994x over previous
import functools

import jax
import jax.numpy as jnp
from jax import lax
from jax.experimental import pallas as pl
from jax.experimental.pallas import tpu as pltpu
from jax.experimental.pallas import tpu_sc as plsc

F = 128
NB = 8
E = 30000
T = 90000
TPAD = 90112
EBLK = 1024
TBLK = 1024
NEB = (E + EBLK - 1) // EBLK
NTB = TPAD // TBLK
NC = 2
NS = 16
CHUNK = 128
NR = 4
EPAD = 30720
ERNG = EPAD // NR
TRASH = CHUNK

_f32 = jnp.float32


def _pre_body(x_ref, rbf_ref, wji_ref, bji_ref, wkj_ref, bkj_ref, wrbf_ref,
              xji_ref, xkj_ref):
    xb = x_ref[...]
    xji = jnp.dot(xb, wji_ref[...], preferred_element_type=_f32) + bji_ref[...]
    xji_ref[...] = jax.nn.silu(xji)
    xkj = jnp.dot(xb, wkj_ref[...], preferred_element_type=_f32) + bkj_ref[...]
    g = jnp.dot(rbf_ref[...], wrbf_ref[...], preferred_element_type=_f32)
    xkj_ref[...] = jax.nn.silu(xkj) * g


def _pre_call(x, rbf, W_ji, b_ji, W_kj, b_kj, W_rbf):
    row = pl.BlockSpec((EBLK, F), lambda i: (i, 0))
    full = lambda shape: pl.BlockSpec(shape, lambda i, s=shape: tuple(0 for _ in s))
    return pl.pallas_call(
        _pre_body,
        grid=(NEB,),
        in_specs=[row,
                  pl.BlockSpec((EBLK, 6), lambda i: (i, 0)),
                  full((F, F)), full((1, F)), full((F, F)), full((1, F)),
                  full((6, F))],
        out_specs=[row, row],
        out_shape=[jax.ShapeDtypeStruct((E, F), _f32),
                   jax.ShapeDtypeStruct((E, F), _f32)],
    )(x, rbf, W_ji, b_ji, W_kj, b_kj, W_rbf)


GC = 64


@functools.cache
def _get_gather_sc():
    return pl.kernel(
        _gather_body,
        out_type=jax.ShapeDtypeStruct((TPAD, F), _f32),
        mesh=plsc.VectorSubcoreMesh(core_axis_name="c", subcore_axis_name="s",
                                    num_cores=NC, num_subcores=NS),
        scratch_types=[pltpu.VMEM((1, GC), jnp.int32),
                       pltpu.VMEM((1, GC), jnp.int32),
                       pltpu.VMEM((GC, F), _f32),
                       pltpu.VMEM((GC, F), _f32),
                       pltpu.SemaphoreType.DMA,
                       pltpu.SemaphoreType.DMA,
                       pltpu.SemaphoreType.DMA,
                       pltpu.SemaphoreType.DMA],
    )


def _gather_body(table_hbm, idx_hbm, out_hbm,
                 idx_a, idx_b, rows_a, rows_b, sga, sgb, soa, sob):
    wid = lax.axis_index("s") * NC + lax.axis_index("c")
    per_w = TPAD // (NC * NS)
    base = wid * per_w

    def body(kk, carry):
        o0 = base + kk * (2 * GC)
        o1 = o0 + GC
        pltpu.sync_copy(idx_hbm.at[pl.ds(o0, GC)], idx_a.at[0])
        ha = pltpu.async_copy(table_hbm.at[idx_a.at[0]], rows_a, sga)
        pltpu.sync_copy(idx_hbm.at[pl.ds(o1, GC)], idx_b.at[0])
        hb = pltpu.async_copy(table_hbm.at[idx_b.at[0]], rows_b, sgb)
        ha.wait()
        hoa = pltpu.async_copy(rows_a, out_hbm.at[pl.ds(o0, GC)], soa)
        hb.wait()
        hob = pltpu.async_copy(rows_b, out_hbm.at[pl.ds(o1, GC)], sob)
        hoa.wait()
        hob.wait()
        return carry

    lax.fori_loop(0, per_w // (2 * GC), body, 0)


def _bilin_body(sbf_ref, g_ref, wsbf_ref, wf_ref, m_ref):
    pid = pl.program_id(0)
    sp = jnp.dot(sbf_ref[...], wsbf_ref[...], preferred_element_type=_f32)
    rows = pid * TBLK + lax.broadcasted_iota(jnp.int32, (TBLK, 1), 0)
    sp = jnp.where(rows < T, sp, 0.0)
    gb = g_ref[...]
    acc = jnp.zeros((TBLK, F), _f32)
    for j in range(NB):
        acc += sp[:, j:j + 1] * jnp.dot(gb, wf_ref[j],
                                        preferred_element_type=_f32)
    m_ref[...] = acc


def _bilin_call(sbf, G, W_sbf, Wf):
    return pl.pallas_call(
        _bilin_body,
        grid=(NTB,),
        in_specs=[pl.BlockSpec((TBLK, 42), lambda i: (i, 0)),
                  pl.BlockSpec((TBLK, F), lambda i: (i, 0)),
                  pl.BlockSpec((42, NB), lambda i: (0, 0)),
                  pl.BlockSpec((NB, F, F), lambda i: (0, 0, 0))],
        out_specs=pl.BlockSpec((TBLK, F), lambda i: (i, 0)),
        out_shape=jax.ShapeDtypeStruct((TPAD, F), _f32),
    )(sbf, G, W_sbf, Wf)


EPS = ERNG // NS
_WCH = ((0, 128), (128, 128), (256, 128), (384, 96))


@functools.cache
def _get_scatter_sc():
    return pl.kernel(
        _scatter_body,
        out_type=jax.ShapeDtypeStruct((EPAD, F), _f32),
        mesh=plsc.VectorSubcoreMesh(core_axis_name="c", subcore_axis_name="s",
                                    num_cores=NC, num_subcores=NS),
        scratch_types=[pltpu.VMEM((TPAD // NS // CHUNK, CHUNK), jnp.int32),
                       pltpu.VMEM((1, CHUNK), jnp.int32),
                       pltpu.VMEM((1, CHUNK), jnp.int32),
                       pltpu.VMEM((CHUNK, F), _f32),
                       pltpu.VMEM((CHUNK, F), _f32),
                       pltpu.VMEM_SHARED((ERNG + TRASH, F), _f32),
                       pltpu.SemaphoreType.DMA,
                       pltpu.SemaphoreType.DMA,
                       pltpu.SemaphoreType.DMA,
                       pltpu.SemaphoreType.DMA],
    )


def _scatter_body(m_hbm, idx3_hbm, zeros_hbm, out_hbm,
                  idxbuf, idx_ta, idx_tb, mbuf_a, mbuf_b, acc,
                  sma, smb, saa, sab):
    c = lax.axis_index("c")
    s = lax.axis_index("s")
    per_s = TPAD // NS
    base = s * per_s
    nchunk = per_s // CHUNK

    pltpu.sync_copy(idx3_hbm.at[s], idxbuf)

    def transform(idx_t, k, e0):
        for j in range(CHUNK // 16):
            v = idxbuf[k, pl.ds(16 * j, 16)] - e0
            trash = ERNG + (16 * j) % TRASH + lax.iota(jnp.int32, 16)
            ok = (v >= 0) & (v < ERNG)
            idx_t[0, pl.ds(16 * j, 16)] = jnp.where(ok, v, trash)

    for p in range(NR // NC):
        r = NC * c + p
        e0 = r * ERNG

        pltpu.sync_copy(zeros_hbm, mbuf_a)
        for off, sz in _WCH:
            pltpu.sync_copy(mbuf_a.at[pl.ds(0, sz)],
                            acc.at[pl.ds(s * EPS + off, sz)])
        plsc.subcore_barrier()

        def body(kk, carry):
            k0 = 2 * kk
            o0 = base + k0 * CHUNK
            ha = pltpu.async_copy(m_hbm.at[pl.ds(o0, CHUNK)], mbuf_a, sma)
            hb = pltpu.async_copy(m_hbm.at[pl.ds(o0 + CHUNK, CHUNK)],
                                  mbuf_b, smb)
            transform(idx_ta, k0, e0)
            transform(idx_tb, k0 + 1, e0)
            ha.wait()
            haa = pltpu.async_copy(mbuf_a, acc.at[idx_ta.at[0]], saa,
                                   add=True)
            hb.wait()
            hab = pltpu.async_copy(mbuf_b, acc.at[idx_tb.at[0]], sab,
                                   add=True)
            haa.wait()
            hab.wait()
            return carry
        lax.fori_loop(0, per_s // (2 * CHUNK), body, 0)
        plsc.subcore_barrier()

        for off, sz in _WCH:
            pltpu.sync_copy(acc.at[pl.ds(s * EPS + off, sz)],
                            mbuf_a.at[pl.ds(0, sz)])
            pltpu.sync_copy(mbuf_a.at[pl.ds(0, sz)],
                            out_hbm.at[pl.ds(e0 + s * EPS + off, sz)])


def _post_body(x_ref, xji_ref, red_ref,
               w1, b1, w2, b2, w3, b3, w4, b4, w5, b5, w6, b6, w7, b7,
               out_ref):
    act = jax.nn.silu

    def lin(v, w, b):
        return jnp.dot(v, w[...], preferred_element_type=_f32) + b[...]

    x2 = xji_ref[...] + red_ref[...]
    h = act(lin(x2, w1, b1))
    h = act(lin(h, w2, b2))
    x2 = x2 + h
    x2 = act(lin(x2, w3, b3))
    xo = x_ref[...] + x2
    h = act(lin(xo, w4, b4))
    h = act(lin(h, w5, b5))
    xo = xo + h
    h = act(lin(xo, w6, b6))
    h = act(lin(h, w7, b7))
    out_ref[...] = xo + h


def _post_call(x, x_ji, red, *wbs):
    row = pl.BlockSpec((EBLK, F), lambda i: (i, 0))
    wspec = pl.BlockSpec((F, F), lambda i: (0, 0))
    bspec = pl.BlockSpec((1, F), lambda i: (0, 0))
    return pl.pallas_call(
        _post_body,
        grid=(NEB,),
        in_specs=[row, row, row] + [wspec, bspec] * 7,
        out_specs=row,
        out_shape=jax.ShapeDtypeStruct((E, F), _f32),
    )(x, x_ji, red, *wbs)


def kernel(x, rbf, sbf, id_expand_kj, id_reduce_ji,
           W_rbf, W_sbf, W_ji, b_ji, W_kj, b_kj, W_bilin,
           W_bs0_0, b_bs0_0, W_bs0_1, b_bs0_1,
           W_fbs, b_fbs,
           W_as0_0, b_as0_0, W_as0_1, b_as0_1,
           W_as1_0, b_as1_0, W_as1_1, b_as1_1):
    b2 = lambda b: b.reshape(1, F)
    x_ji, x_kj = _pre_call(x, rbf, W_ji, b2(b_ji), W_kj, b2(b_kj), W_rbf)

    ide = jnp.pad(id_expand_kj.astype(jnp.int32), (0, TPAD - T))
    G = _get_gather_sc()(x_kj, ide)

    Wf = jnp.transpose(W_bilin, (1, 2, 0))
    m = _bilin_call(sbf, G, W_sbf, Wf)

    idr = jnp.pad(id_reduce_ji.astype(jnp.int32), (0, TPAD - T))
    idr3 = idr.reshape(NS, TPAD // NS // CHUNK, CHUNK)
    zeros = jnp.zeros((CHUNK, F), _f32)
    red = _get_scatter_sc()(m, idr3, zeros)

    return _post_call(x, x_ji, red,
                      W_bs0_0, b2(b_bs0_0), W_bs0_1, b2(b_bs0_1),
                      W_fbs, b2(b_fbs),
                      W_as0_0, b2(b_as0_0), W_as0_1, b2(b_as0_1),
                      W_as1_0, b2(b_as1_0), W_as1_1, b2(b_as1_1))


# speedup vs baseline: 2.6229x; 1.0876x over previous
import functools

import jax
import jax.numpy as jnp
from jax import lax
from jax.experimental import pallas as pl
from jax.experimental.pallas import tpu as pltpu
from jax.experimental.pallas import tpu_sc as plsc

F = 128
NB = 8
E = 30000
T = 90000
TPAD = 90112
EBLK = 2048
TBLK = 2048
NEB = (E + EBLK - 1) // EBLK
NTB = TPAD // TBLK
NC = 2
NS = 16
CHUNK = 128
NR = 4
EPAD = 30720
ERNG = EPAD // NR
TRASH = CHUNK

_f32 = jnp.float32


def _pre_body(x_ref, rbf_ref, wji_ref, bji_ref, wkj_ref, bkj_ref, wrbf_ref,
              xji_ref, xkj_ref):
    xb = x_ref[...]
    xji = jnp.dot(xb, wji_ref[...], preferred_element_type=_f32) + bji_ref[...]
    xji_ref[...] = jax.nn.silu(xji)
    xkj = jnp.dot(xb, wkj_ref[...], preferred_element_type=_f32) + bkj_ref[...]
    g = jnp.dot(rbf_ref[...], wrbf_ref[...], preferred_element_type=_f32)
    xkj_ref[...] = jax.nn.silu(xkj) * g


def _pre_call(x, rbf, W_ji, b_ji, W_kj, b_kj, W_rbf):
    row = pl.BlockSpec((EBLK, F), lambda i: (i, 0))
    full = lambda shape: pl.BlockSpec(shape, lambda i, s=shape: tuple(0 for _ in s))
    return pl.pallas_call(
        _pre_body,
        grid=(NEB,),
        in_specs=[row,
                  pl.BlockSpec((EBLK, 6), lambda i: (i, 0)),
                  full((F, F)), full((1, F)), full((F, F)), full((1, F)),
                  full((6, F))],
        out_specs=[row, row],
        out_shape=[jax.ShapeDtypeStruct((E, F), _f32),
                   jax.ShapeDtypeStruct((E, F), _f32)],
    )(x, rbf, W_ji, b_ji, W_kj, b_kj, W_rbf)


GC = 64


@functools.cache
def _get_gather_sc():
    return pl.kernel(
        _gather_body,
        out_type=jax.ShapeDtypeStruct((TPAD, F), _f32),
        mesh=plsc.VectorSubcoreMesh(core_axis_name="c", subcore_axis_name="s",
                                    num_cores=NC, num_subcores=NS),
        scratch_types=[pltpu.VMEM((1, GC), jnp.int32),
                       pltpu.VMEM((1, GC), jnp.int32),
                       pltpu.VMEM((GC, F), _f32),
                       pltpu.VMEM((GC, F), _f32),
                       pltpu.SemaphoreType.DMA,
                       pltpu.SemaphoreType.DMA,
                       pltpu.SemaphoreType.DMA,
                       pltpu.SemaphoreType.DMA],
    )


def _gather_body(table_hbm, idx_hbm, out_hbm,
                 idx_a, idx_b, rows_a, rows_b, sga, sgb, soa, sob):
    wid = lax.axis_index("s") * NC + lax.axis_index("c")
    per_w = TPAD // (NC * NS)
    base = wid * per_w

    def body(kk, carry):
        o0 = base + kk * (2 * GC)
        o1 = o0 + GC
        pltpu.sync_copy(idx_hbm.at[pl.ds(o0, GC)], idx_a.at[0])
        ha = pltpu.async_copy(table_hbm.at[idx_a.at[0]], rows_a, sga)
        pltpu.sync_copy(idx_hbm.at[pl.ds(o1, GC)], idx_b.at[0])
        hb = pltpu.async_copy(table_hbm.at[idx_b.at[0]], rows_b, sgb)
        ha.wait()
        hoa = pltpu.async_copy(rows_a, out_hbm.at[pl.ds(o0, GC)], soa)
        hb.wait()
        hob = pltpu.async_copy(rows_b, out_hbm.at[pl.ds(o1, GC)], sob)
        hoa.wait()
        hob.wait()
        return carry

    lax.fori_loop(0, per_w // (2 * GC), body, 0)


def _bilin_body(sbf_ref, g_ref, wsbf_ref, wf_ref, m_ref):
    pid = pl.program_id(0)
    sp = jnp.dot(sbf_ref[...], wsbf_ref[...], preferred_element_type=_f32)
    rows = pid * TBLK + lax.broadcasted_iota(jnp.int32, (TBLK, 1), 0)
    sp = jnp.where(rows < T, sp, 0.0)
    gb = g_ref[...]
    acc = jnp.zeros((TBLK, F), _f32)
    for j in range(NB):
        acc += sp[:, j:j + 1] * jnp.dot(gb, wf_ref[j],
                                        preferred_element_type=_f32)
    m_ref[...] = acc


def _bilin_call(sbf, G, W_sbf, Wf):
    return pl.pallas_call(
        _bilin_body,
        grid=(NTB,),
        in_specs=[pl.BlockSpec((TBLK, 42), lambda i: (i, 0)),
                  pl.BlockSpec((TBLK, F), lambda i: (i, 0)),
                  pl.BlockSpec((42, NB), lambda i: (0, 0)),
                  pl.BlockSpec((NB, F, F), lambda i: (0, 0, 0))],
        out_specs=pl.BlockSpec((TBLK, F), lambda i: (i, 0)),
        out_shape=jax.ShapeDtypeStruct((TPAD, F), _f32),
    )(sbf, G, W_sbf, Wf)


EPS = ERNG // NS
_WCH = ((0, 128), (128, 128), (256, 128), (384, 96))


@functools.cache
def _get_scatter_sc():
    return pl.kernel(
        _scatter_body,
        out_type=jax.ShapeDtypeStruct((EPAD, F), _f32),
        mesh=plsc.VectorSubcoreMesh(core_axis_name="c", subcore_axis_name="s",
                                    num_cores=NC, num_subcores=NS),
        scratch_types=[pltpu.VMEM((TPAD // NS // CHUNK, CHUNK), jnp.int32),
                       pltpu.VMEM((1, CHUNK), jnp.int32),
                       pltpu.VMEM((1, CHUNK), jnp.int32),
                       pltpu.VMEM((CHUNK, F), _f32),
                       pltpu.VMEM((CHUNK, F), _f32),
                       pltpu.VMEM_SHARED((ERNG + TRASH, F), _f32),
                       pltpu.SemaphoreType.DMA,
                       pltpu.SemaphoreType.DMA,
                       pltpu.SemaphoreType.DMA,
                       pltpu.SemaphoreType.DMA],
    )


def _scatter_body(m_hbm, idx3_hbm, zeros_hbm, out_hbm,
                  idxbuf, idx_ta, idx_tb, mbuf_a, mbuf_b, acc,
                  sma, smb, saa, sab):
    c = lax.axis_index("c")
    s = lax.axis_index("s")
    per_s = TPAD // NS
    base = s * per_s
    nchunk = per_s // CHUNK

    pltpu.sync_copy(idx3_hbm.at[s], idxbuf)

    def transform(idx_t, k, e0):
        for j in range(CHUNK // 16):
            v = idxbuf[k, pl.ds(16 * j, 16)] - e0
            trash = ERNG + (16 * j) % TRASH + lax.iota(jnp.int32, 16)
            ok = (v >= 0) & (v < ERNG)
            idx_t[0, pl.ds(16 * j, 16)] = jnp.where(ok, v, trash)

    for p in range(NR // NC):
        r = NC * c + p
        e0 = r * ERNG

        pltpu.sync_copy(zeros_hbm, mbuf_a)
        for off, sz in _WCH:
            pltpu.sync_copy(mbuf_a.at[pl.ds(0, sz)],
                            acc.at[pl.ds(s * EPS + off, sz)])
        plsc.subcore_barrier()

        def body(kk, carry):
            k0 = 2 * kk
            o0 = base + k0 * CHUNK
            ha = pltpu.async_copy(m_hbm.at[pl.ds(o0, CHUNK)], mbuf_a, sma)
            hb = pltpu.async_copy(m_hbm.at[pl.ds(o0 + CHUNK, CHUNK)],
                                  mbuf_b, smb)
            transform(idx_ta, k0, e0)
            transform(idx_tb, k0 + 1, e0)
            ha.wait()
            haa = pltpu.async_copy(mbuf_a, acc.at[idx_ta.at[0]], saa,
                                   add=True)
            hb.wait()
            hab = pltpu.async_copy(mbuf_b, acc.at[idx_tb.at[0]], sab,
                                   add=True)
            haa.wait()
            hab.wait()
            return carry
        lax.fori_loop(0, per_s // (2 * CHUNK), body, 0)
        plsc.subcore_barrier()

        for off, sz in _WCH:
            pltpu.sync_copy(acc.at[pl.ds(s * EPS + off, sz)],
                            mbuf_a.at[pl.ds(0, sz)])
            pltpu.sync_copy(mbuf_a.at[pl.ds(0, sz)],
                            out_hbm.at[pl.ds(e0 + s * EPS + off, sz)])


def _post_body(x_ref, xji_ref, red_ref,
               w1, b1, w2, b2, w3, b3, w4, b4, w5, b5, w6, b6, w7, b7,
               out_ref):
    act = jax.nn.silu

    def lin(v, w, b):
        return jnp.dot(v, w[...], preferred_element_type=_f32) + b[...]

    x2 = xji_ref[...] + red_ref[...]
    h = act(lin(x2, w1, b1))
    h = act(lin(h, w2, b2))
    x2 = x2 + h
    x2 = act(lin(x2, w3, b3))
    xo = x_ref[...] + x2
    h = act(lin(xo, w4, b4))
    h = act(lin(h, w5, b5))
    xo = xo + h
    h = act(lin(xo, w6, b6))
    h = act(lin(h, w7, b7))
    out_ref[...] = xo + h


def _post_call(x, x_ji, red, *wbs):
    row = pl.BlockSpec((EBLK, F), lambda i: (i, 0))
    wspec = pl.BlockSpec((F, F), lambda i: (0, 0))
    bspec = pl.BlockSpec((1, F), lambda i: (0, 0))
    return pl.pallas_call(
        _post_body,
        grid=(NEB,),
        in_specs=[row, row, row] + [wspec, bspec] * 7,
        out_specs=row,
        out_shape=jax.ShapeDtypeStruct((E, F), _f32),
    )(x, x_ji, red, *wbs)


def kernel(x, rbf, sbf, id_expand_kj, id_reduce_ji,
           W_rbf, W_sbf, W_ji, b_ji, W_kj, b_kj, W_bilin,
           W_bs0_0, b_bs0_0, W_bs0_1, b_bs0_1,
           W_fbs, b_fbs,
           W_as0_0, b_as0_0, W_as0_1, b_as0_1,
           W_as1_0, b_as1_0, W_as1_1, b_as1_1):
    b2 = lambda b: b.reshape(1, F)
    x_ji, x_kj = _pre_call(x, rbf, W_ji, b2(b_ji), W_kj, b2(b_kj), W_rbf)

    ide = jnp.pad(id_expand_kj.astype(jnp.int32), (0, TPAD - T))
    G = _get_gather_sc()(x_kj, ide)

    Wf = jnp.transpose(W_bilin, (1, 2, 0))
    m = _bilin_call(sbf, G, W_sbf, Wf)

    idr = jnp.pad(id_reduce_ji.astype(jnp.int32), (0, TPAD - T))
    idr3 = idr.reshape(NS, TPAD // NS // CHUNK, CHUNK)
    zeros = jnp.zeros((CHUNK, F), _f32)
    red = _get_scatter_sc()(m, idr3, zeros)

    return _post_call(x, x_ji, red,
                      W_bs0_0, b2(b_bs0_0), W_bs0_1, b2(b_bs0_1),
                      W_fbs, b2(b_fbs),
                      W_as0_0, b2(b_as0_0), W_as0_1, b2(b_as0_1),
                      W_as1_0, b2(b_as1_0), W_as1_1, b2(b_as1_1))


# speedup vs baseline: 2.6902x; 1.0257x over previous
import functools

import jax
import jax.numpy as jnp
from jax import lax
from jax.experimental import pallas as pl
from jax.experimental.pallas import tpu as pltpu
from jax.experimental.pallas import tpu_sc as plsc

F = 128
NB = 8
E = 30000
T = 90000
TPAD = 90112
EBLK = 4096
TBLK = 4096
NEB = (E + EBLK - 1) // EBLK
NTB = TPAD // TBLK
NC = 2
NS = 16
CHUNK = 128
NR = 4
EPAD = 30720
ERNG = EPAD // NR
TRASH = CHUNK

_f32 = jnp.float32


def _pre_body(x_ref, rbf_ref, wji_ref, bji_ref, wkj_ref, bkj_ref, wrbf_ref,
              xji_ref, xkj_ref):
    xb = x_ref[...]
    xji = jnp.dot(xb, wji_ref[...], preferred_element_type=_f32) + bji_ref[...]
    xji_ref[...] = jax.nn.silu(xji)
    xkj = jnp.dot(xb, wkj_ref[...], preferred_element_type=_f32) + bkj_ref[...]
    g = jnp.dot(rbf_ref[...], wrbf_ref[...], preferred_element_type=_f32)
    xkj_ref[...] = jax.nn.silu(xkj) * g


def _pre_call(x, rbf, W_ji, b_ji, W_kj, b_kj, W_rbf):
    row = pl.BlockSpec((EBLK, F), lambda i: (i, 0))
    full = lambda shape: pl.BlockSpec(shape, lambda i, s=shape: tuple(0 for _ in s))
    return pl.pallas_call(
        _pre_body,
        grid=(NEB,),
        in_specs=[row,
                  pl.BlockSpec((EBLK, 6), lambda i: (i, 0)),
                  full((F, F)), full((1, F)), full((F, F)), full((1, F)),
                  full((6, F))],
        out_specs=[row, row],
        out_shape=[jax.ShapeDtypeStruct((E, F), _f32),
                   jax.ShapeDtypeStruct((E, F), _f32)],
    )(x, rbf, W_ji, b_ji, W_kj, b_kj, W_rbf)


GC = 64


@functools.cache
def _get_gather_sc():
    return pl.kernel(
        _gather_body,
        out_type=jax.ShapeDtypeStruct((TPAD, F), _f32),
        mesh=plsc.VectorSubcoreMesh(core_axis_name="c", subcore_axis_name="s",
                                    num_cores=NC, num_subcores=NS),
        scratch_types=[pltpu.VMEM((1, GC), jnp.int32),
                       pltpu.VMEM((1, GC), jnp.int32),
                       pltpu.VMEM((GC, F), _f32),
                       pltpu.VMEM((GC, F), _f32),
                       pltpu.SemaphoreType.DMA,
                       pltpu.SemaphoreType.DMA,
                       pltpu.SemaphoreType.DMA,
                       pltpu.SemaphoreType.DMA],
    )


def _gather_body(table_hbm, idx_hbm, out_hbm,
                 idx_a, idx_b, rows_a, rows_b, sga, sgb, soa, sob):
    wid = lax.axis_index("s") * NC + lax.axis_index("c")
    per_w = TPAD // (NC * NS)
    base = wid * per_w

    def body(kk, carry):
        o0 = base + kk * (2 * GC)
        o1 = o0 + GC
        pltpu.sync_copy(idx_hbm.at[pl.ds(o0, GC)], idx_a.at[0])
        ha = pltpu.async_copy(table_hbm.at[idx_a.at[0]], rows_a, sga)
        pltpu.sync_copy(idx_hbm.at[pl.ds(o1, GC)], idx_b.at[0])
        hb = pltpu.async_copy(table_hbm.at[idx_b.at[0]], rows_b, sgb)
        ha.wait()
        hoa = pltpu.async_copy(rows_a, out_hbm.at[pl.ds(o0, GC)], soa)
        hb.wait()
        hob = pltpu.async_copy(rows_b, out_hbm.at[pl.ds(o1, GC)], sob)
        hoa.wait()
        hob.wait()
        return carry

    lax.fori_loop(0, per_w // (2 * GC), body, 0)


def _bilin_body(sbf_ref, g_ref, wsbf_ref, wf_ref, m_ref):
    pid = pl.program_id(0)
    sp = jnp.dot(sbf_ref[...], wsbf_ref[...], preferred_element_type=_f32)
    rows = pid * TBLK + lax.broadcasted_iota(jnp.int32, (TBLK, 1), 0)
    sp = jnp.where(rows < T, sp, 0.0)
    gb = g_ref[...]
    acc = jnp.zeros((TBLK, F), _f32)
    for j in range(NB):
        acc += sp[:, j:j + 1] * jnp.dot(gb, wf_ref[j],
                                        preferred_element_type=_f32)
    m_ref[...] = acc


def _bilin_call(sbf, G, W_sbf, Wf):
    return pl.pallas_call(
        _bilin_body,
        grid=(NTB,),
        in_specs=[pl.BlockSpec((TBLK, 42), lambda i: (i, 0)),
                  pl.BlockSpec((TBLK, F), lambda i: (i, 0)),
                  pl.BlockSpec((42, NB), lambda i: (0, 0)),
                  pl.BlockSpec((NB, F, F), lambda i: (0, 0, 0))],
        out_specs=pl.BlockSpec((TBLK, F), lambda i: (i, 0)),
        out_shape=jax.ShapeDtypeStruct((TPAD, F), _f32),
    )(sbf, G, W_sbf, Wf)


EPS = ERNG // NS
_WCH = ((0, 128), (128, 128), (256, 128), (384, 96))


@functools.cache
def _get_scatter_sc():
    return pl.kernel(
        _scatter_body,
        out_type=jax.ShapeDtypeStruct((EPAD, F), _f32),
        mesh=plsc.VectorSubcoreMesh(core_axis_name="c", subcore_axis_name="s",
                                    num_cores=NC, num_subcores=NS),
        scratch_types=[pltpu.VMEM((TPAD // NS // CHUNK, CHUNK), jnp.int32),
                       pltpu.VMEM((1, CHUNK), jnp.int32),
                       pltpu.VMEM((1, CHUNK), jnp.int32),
                       pltpu.VMEM((CHUNK, F), _f32),
                       pltpu.VMEM((CHUNK, F), _f32),
                       pltpu.VMEM_SHARED((ERNG + TRASH, F), _f32),
                       pltpu.SemaphoreType.DMA,
                       pltpu.SemaphoreType.DMA,
                       pltpu.SemaphoreType.DMA,
                       pltpu.SemaphoreType.DMA],
    )


def _scatter_body(m_hbm, idx3_hbm, zeros_hbm, out_hbm,
                  idxbuf, idx_ta, idx_tb, mbuf_a, mbuf_b, acc,
                  sma, smb, saa, sab):
    c = lax.axis_index("c")
    s = lax.axis_index("s")
    per_s = TPAD // NS
    base = s * per_s
    nchunk = per_s // CHUNK

    pltpu.sync_copy(idx3_hbm.at[s], idxbuf)

    def transform(idx_t, k, e0):
        for j in range(CHUNK // 16):
            v = idxbuf[k, pl.ds(16 * j, 16)] - e0
            trash = ERNG + (16 * j) % TRASH + lax.iota(jnp.int32, 16)
            ok = (v >= 0) & (v < ERNG)
            idx_t[0, pl.ds(16 * j, 16)] = jnp.where(ok, v, trash)

    for p in range(NR // NC):
        r = NC * c + p
        e0 = r * ERNG

        pltpu.sync_copy(zeros_hbm, mbuf_a)
        for off, sz in _WCH:
            pltpu.sync_copy(mbuf_a.at[pl.ds(0, sz)],
                            acc.at[pl.ds(s * EPS + off, sz)])
        plsc.subcore_barrier()

        def body(kk, carry):
            k0 = 2 * kk
            o0 = base + k0 * CHUNK
            ha = pltpu.async_copy(m_hbm.at[pl.ds(o0, CHUNK)], mbuf_a, sma)
            hb = pltpu.async_copy(m_hbm.at[pl.ds(o0 + CHUNK, CHUNK)],
                                  mbuf_b, smb)
            transform(idx_ta, k0, e0)
            transform(idx_tb, k0 + 1, e0)
            ha.wait()
            haa = pltpu.async_copy(mbuf_a, acc.at[idx_ta.at[0]], saa,
                                   add=True)
            hb.wait()
            hab = pltpu.async_copy(mbuf_b, acc.at[idx_tb.at[0]], sab,
                                   add=True)
            haa.wait()
            hab.wait()
            return carry
        lax.fori_loop(0, per_s // (2 * CHUNK), body, 0)
        plsc.subcore_barrier()

        for off, sz in _WCH:
            pltpu.sync_copy(acc.at[pl.ds(s * EPS + off, sz)],
                            mbuf_a.at[pl.ds(0, sz)])
            pltpu.sync_copy(mbuf_a.at[pl.ds(0, sz)],
                            out_hbm.at[pl.ds(e0 + s * EPS + off, sz)])


def _post_body(x_ref, xji_ref, red_ref,
               w1, b1, w2, b2, w3, b3, w4, b4, w5, b5, w6, b6, w7, b7,
               out_ref):
    act = jax.nn.silu

    def lin(v, w, b):
        return jnp.dot(v, w[...], preferred_element_type=_f32) + b[...]

    x2 = xji_ref[...] + red_ref[...]
    h = act(lin(x2, w1, b1))
    h = act(lin(h, w2, b2))
    x2 = x2 + h
    x2 = act(lin(x2, w3, b3))
    xo = x_ref[...] + x2
    h = act(lin(xo, w4, b4))
    h = act(lin(h, w5, b5))
    xo = xo + h
    h = act(lin(xo, w6, b6))
    h = act(lin(h, w7, b7))
    out_ref[...] = xo + h


def _post_call(x, x_ji, red, *wbs):
    row = pl.BlockSpec((EBLK, F), lambda i: (i, 0))
    wspec = pl.BlockSpec((F, F), lambda i: (0, 0))
    bspec = pl.BlockSpec((1, F), lambda i: (0, 0))
    return pl.pallas_call(
        _post_body,
        grid=(NEB,),
        in_specs=[row, row, row] + [wspec, bspec] * 7,
        out_specs=row,
        out_shape=jax.ShapeDtypeStruct((E, F), _f32),
    )(x, x_ji, red, *wbs)


def kernel(x, rbf, sbf, id_expand_kj, id_reduce_ji,
           W_rbf, W_sbf, W_ji, b_ji, W_kj, b_kj, W_bilin,
           W_bs0_0, b_bs0_0, W_bs0_1, b_bs0_1,
           W_fbs, b_fbs,
           W_as0_0, b_as0_0, W_as0_1, b_as0_1,
           W_as1_0, b_as1_0, W_as1_1, b_as1_1):
    b2 = lambda b: b.reshape(1, F)
    x_ji, x_kj = _pre_call(x, rbf, W_ji, b2(b_ji), W_kj, b2(b_kj), W_rbf)

    ide = jnp.pad(id_expand_kj.astype(jnp.int32), (0, TPAD - T))
    G = _get_gather_sc()(x_kj, ide)

    Wf = jnp.transpose(W_bilin, (1, 2, 0))
    m = _bilin_call(sbf, G, W_sbf, Wf)

    idr = jnp.pad(id_reduce_ji.astype(jnp.int32), (0, TPAD - T))
    idr3 = idr.reshape(NS, TPAD // NS // CHUNK, CHUNK)
    zeros = jnp.zeros((CHUNK, F), _f32)
    red = _get_scatter_sc()(m, idr3, zeros)

    return _post_call(x, x_ji, red,
                      W_bs0_0, b2(b_bs0_0), W_bs0_1, b2(b_bs0_1),
                      W_fbs, b2(b_fbs),
                      W_as0_0, b2(b_as0_0), W_as0_1, b2(b_as0_1),
                      W_as1_0, b2(b_as1_0), W_as1_1, b2(b_as1_1))


# speedup vs baseline: 2.7174x; 1.0101x over previous
import functools

import jax
import jax.numpy as jnp
from jax import lax
from jax.experimental import pallas as pl
from jax.experimental.pallas import tpu as pltpu
from jax.experimental.pallas import tpu_sc as plsc

F = 128
NB = 8
E = 30000
T = 90000
TPAD = 90112
EBLK = 7680
TBLK = 8192
NEB = (E + EBLK - 1) // EBLK
NTB = TPAD // TBLK
NC = 2
NS = 16
CHUNK = 128
NR = 4
EPAD = 30720
ERNG = EPAD // NR
TRASH = CHUNK

_f32 = jnp.float32


def _pre_body(x_ref, rbf_ref, wji_ref, bji_ref, wkj_ref, bkj_ref, wrbf_ref,
              xji_ref, xkj_ref):
    xb = x_ref[...]
    xji = jnp.dot(xb, wji_ref[...], preferred_element_type=_f32) + bji_ref[...]
    xji_ref[...] = jax.nn.silu(xji)
    xkj = jnp.dot(xb, wkj_ref[...], preferred_element_type=_f32) + bkj_ref[...]
    g = jnp.dot(rbf_ref[...], wrbf_ref[...], preferred_element_type=_f32)
    xkj_ref[...] = jax.nn.silu(xkj) * g


def _pre_call(x, rbf, W_ji, b_ji, W_kj, b_kj, W_rbf):
    row = pl.BlockSpec((EBLK, F), lambda i: (i, 0))
    full = lambda shape: pl.BlockSpec(shape, lambda i, s=shape: tuple(0 for _ in s))
    return pl.pallas_call(
        _pre_body,
        grid=(NEB,),
        in_specs=[row,
                  pl.BlockSpec((EBLK, 6), lambda i: (i, 0)),
                  full((F, F)), full((1, F)), full((F, F)), full((1, F)),
                  full((6, F))],
        out_specs=[row, row],
        out_shape=[jax.ShapeDtypeStruct((E, F), _f32),
                   jax.ShapeDtypeStruct((E, F), _f32)],
    )(x, rbf, W_ji, b_ji, W_kj, b_kj, W_rbf)


GC = 64


@functools.cache
def _get_gather_sc():
    return pl.kernel(
        _gather_body,
        out_type=jax.ShapeDtypeStruct((TPAD, F), _f32),
        mesh=plsc.VectorSubcoreMesh(core_axis_name="c", subcore_axis_name="s",
                                    num_cores=NC, num_subcores=NS),
        scratch_types=[pltpu.VMEM((1, GC), jnp.int32),
                       pltpu.VMEM((1, GC), jnp.int32),
                       pltpu.VMEM((GC, F), _f32),
                       pltpu.VMEM((GC, F), _f32),
                       pltpu.SemaphoreType.DMA,
                       pltpu.SemaphoreType.DMA,
                       pltpu.SemaphoreType.DMA,
                       pltpu.SemaphoreType.DMA],
    )


def _gather_body(table_hbm, idx_hbm, out_hbm,
                 idx_a, idx_b, rows_a, rows_b, sga, sgb, soa, sob):
    wid = lax.axis_index("s") * NC + lax.axis_index("c")
    per_w = TPAD // (NC * NS)
    base = wid * per_w

    def body(kk, carry):
        o0 = base + kk * (2 * GC)
        o1 = o0 + GC
        pltpu.sync_copy(idx_hbm.at[pl.ds(o0, GC)], idx_a.at[0])
        ha = pltpu.async_copy(table_hbm.at[idx_a.at[0]], rows_a, sga)
        pltpu.sync_copy(idx_hbm.at[pl.ds(o1, GC)], idx_b.at[0])
        hb = pltpu.async_copy(table_hbm.at[idx_b.at[0]], rows_b, sgb)
        ha.wait()
        hoa = pltpu.async_copy(rows_a, out_hbm.at[pl.ds(o0, GC)], soa)
        hb.wait()
        hob = pltpu.async_copy(rows_b, out_hbm.at[pl.ds(o1, GC)], sob)
        hoa.wait()
        hob.wait()
        return carry

    lax.fori_loop(0, per_w // (2 * GC), body, 0)


def _bilin_body(sbf_ref, g_ref, wsbf_ref, wf_ref, m_ref):
    pid = pl.program_id(0)
    sp = jnp.dot(sbf_ref[...], wsbf_ref[...], preferred_element_type=_f32)
    rows = pid * TBLK + lax.broadcasted_iota(jnp.int32, (TBLK, 1), 0)
    sp = jnp.where(rows < T, sp, 0.0)
    gb = g_ref[...]
    acc = jnp.zeros((TBLK, F), _f32)
    for j in range(NB):
        acc += sp[:, j:j + 1] * jnp.dot(gb, wf_ref[j],
                                        preferred_element_type=_f32)
    m_ref[...] = acc


def _bilin_call(sbf, G, W_sbf, Wf):
    return pl.pallas_call(
        _bilin_body,
        grid=(NTB,),
        in_specs=[pl.BlockSpec((TBLK, 42), lambda i: (i, 0)),
                  pl.BlockSpec((TBLK, F), lambda i: (i, 0)),
                  pl.BlockSpec((42, NB), lambda i: (0, 0)),
                  pl.BlockSpec((NB, F, F), lambda i: (0, 0, 0))],
        out_specs=pl.BlockSpec((TBLK, F), lambda i: (i, 0)),
        out_shape=jax.ShapeDtypeStruct((TPAD, F), _f32),
    )(sbf, G, W_sbf, Wf)


EPS = ERNG // NS
_WCH = ((0, 128), (128, 128), (256, 128), (384, 96))


@functools.cache
def _get_scatter_sc():
    return pl.kernel(
        _scatter_body,
        out_type=jax.ShapeDtypeStruct((EPAD, F), _f32),
        mesh=plsc.VectorSubcoreMesh(core_axis_name="c", subcore_axis_name="s",
                                    num_cores=NC, num_subcores=NS),
        scratch_types=[pltpu.VMEM((TPAD // NS // CHUNK, CHUNK), jnp.int32),
                       pltpu.VMEM((1, CHUNK), jnp.int32),
                       pltpu.VMEM((1, CHUNK), jnp.int32),
                       pltpu.VMEM((CHUNK, F), _f32),
                       pltpu.VMEM((CHUNK, F), _f32),
                       pltpu.VMEM_SHARED((ERNG + TRASH, F), _f32),
                       pltpu.SemaphoreType.DMA,
                       pltpu.SemaphoreType.DMA,
                       pltpu.SemaphoreType.DMA,
                       pltpu.SemaphoreType.DMA],
    )


def _scatter_body(m_hbm, idx3_hbm, zeros_hbm, out_hbm,
                  idxbuf, idx_ta, idx_tb, mbuf_a, mbuf_b, acc,
                  sma, smb, saa, sab):
    c = lax.axis_index("c")
    s = lax.axis_index("s")
    per_s = TPAD // NS
    base = s * per_s
    nchunk = per_s // CHUNK

    pltpu.sync_copy(idx3_hbm.at[s], idxbuf)

    def transform(idx_t, k, e0):
        for j in range(CHUNK // 16):
            v = idxbuf[k, pl.ds(16 * j, 16)] - e0
            trash = ERNG + (16 * j) % TRASH + lax.iota(jnp.int32, 16)
            ok = (v >= 0) & (v < ERNG)
            idx_t[0, pl.ds(16 * j, 16)] = jnp.where(ok, v, trash)

    for p in range(NR // NC):
        r = NC * c + p
        e0 = r * ERNG

        pltpu.sync_copy(zeros_hbm, mbuf_a)
        for off, sz in _WCH:
            pltpu.sync_copy(mbuf_a.at[pl.ds(0, sz)],
                            acc.at[pl.ds(s * EPS + off, sz)])
        plsc.subcore_barrier()

        def body(kk, carry):
            k0 = 2 * kk
            o0 = base + k0 * CHUNK
            ha = pltpu.async_copy(m_hbm.at[pl.ds(o0, CHUNK)], mbuf_a, sma)
            hb = pltpu.async_copy(m_hbm.at[pl.ds(o0 + CHUNK, CHUNK)],
                                  mbuf_b, smb)
            transform(idx_ta, k0, e0)
            transform(idx_tb, k0 + 1, e0)
            ha.wait()
            haa = pltpu.async_copy(mbuf_a, acc.at[idx_ta.at[0]], saa,
                                   add=True)
            hb.wait()
            hab = pltpu.async_copy(mbuf_b, acc.at[idx_tb.at[0]], sab,
                                   add=True)
            haa.wait()
            hab.wait()
            return carry
        lax.fori_loop(0, per_s // (2 * CHUNK), body, 0)
        plsc.subcore_barrier()

        for off, sz in _WCH:
            pltpu.sync_copy(acc.at[pl.ds(s * EPS + off, sz)],
                            mbuf_a.at[pl.ds(0, sz)])
            pltpu.sync_copy(mbuf_a.at[pl.ds(0, sz)],
                            out_hbm.at[pl.ds(e0 + s * EPS + off, sz)])


def _post_body(x_ref, xji_ref, red_ref,
               w1, b1, w2, b2, w3, b3, w4, b4, w5, b5, w6, b6, w7, b7,
               out_ref):
    act = jax.nn.silu

    def lin(v, w, b):
        return jnp.dot(v, w[...], preferred_element_type=_f32) + b[...]

    x2 = xji_ref[...] + red_ref[...]
    h = act(lin(x2, w1, b1))
    h = act(lin(h, w2, b2))
    x2 = x2 + h
    x2 = act(lin(x2, w3, b3))
    xo = x_ref[...] + x2
    h = act(lin(xo, w4, b4))
    h = act(lin(h, w5, b5))
    xo = xo + h
    h = act(lin(xo, w6, b6))
    h = act(lin(h, w7, b7))
    out_ref[...] = xo + h


def _post_call(x, x_ji, red, *wbs):
    row = pl.BlockSpec((EBLK, F), lambda i: (i, 0))
    wspec = pl.BlockSpec((F, F), lambda i: (0, 0))
    bspec = pl.BlockSpec((1, F), lambda i: (0, 0))
    return pl.pallas_call(
        _post_body,
        grid=(NEB,),
        in_specs=[row, row, row] + [wspec, bspec] * 7,
        out_specs=row,
        out_shape=jax.ShapeDtypeStruct((E, F), _f32),
    )(x, x_ji, red, *wbs)


def kernel(x, rbf, sbf, id_expand_kj, id_reduce_ji,
           W_rbf, W_sbf, W_ji, b_ji, W_kj, b_kj, W_bilin,
           W_bs0_0, b_bs0_0, W_bs0_1, b_bs0_1,
           W_fbs, b_fbs,
           W_as0_0, b_as0_0, W_as0_1, b_as0_1,
           W_as1_0, b_as1_0, W_as1_1, b_as1_1):
    b2 = lambda b: b.reshape(1, F)
    x_ji, x_kj = _pre_call(x, rbf, W_ji, b2(b_ji), W_kj, b2(b_kj), W_rbf)

    ide = jnp.pad(id_expand_kj.astype(jnp.int32), (0, TPAD - T))
    G = _get_gather_sc()(x_kj, ide)

    Wf = jnp.transpose(W_bilin, (1, 2, 0))
    m = _bilin_call(sbf, G, W_sbf, Wf)

    idr = jnp.pad(id_reduce_ji.astype(jnp.int32), (0, TPAD - T))
    idr3 = idr.reshape(NS, TPAD // NS // CHUNK, CHUNK)
    zeros = jnp.zeros((CHUNK, F), _f32)
    red = _get_scatter_sc()(m, idr3, zeros)

    return _post_call(x, x_ji, red,
                      W_bs0_0, b2(b_bs0_0), W_bs0_1, b2(b_bs0_1),
                      W_fbs, b2(b_fbs),
                      W_as0_0, b2(b_as0_0), W_as0_1, b2(b_as0_1),
                      W_as1_0, b2(b_as1_0), W_as1_1, b2(b_as1_1))


# speedup vs baseline: 2.8420x; 1.0458x over previous
import functools

import jax
import jax.numpy as jnp
from jax import lax
from jax.experimental import pallas as pl
from jax.experimental.pallas import tpu as pltpu
from jax.experimental.pallas import tpu_sc as plsc

F = 128
NB = 8
E = 30000
T = 90000
TPAD = 90112
EBLK = 7680
TBLK = 4096
TH = TPAD // 2
NEB = (E + EBLK - 1) // EBLK
NTB = TPAD // TBLK
NC = 2
NS = 16
CHUNK = 128
NR = 4
EPAD = 30720
ERNG = EPAD // NR
TRASH = CHUNK

_f32 = jnp.float32


def _pre_body(x_ref, rbf_ref, wji_ref, bji_ref, wkj_ref, bkj_ref, wrbf_ref,
              xji_ref, xkj_ref):
    xb = x_ref[...]
    xji = jnp.dot(xb, wji_ref[...], preferred_element_type=_f32) + bji_ref[...]
    xji_ref[...] = jax.nn.silu(xji)
    xkj = jnp.dot(xb, wkj_ref[...], preferred_element_type=_f32) + bkj_ref[...]
    g = jnp.dot(rbf_ref[...], wrbf_ref[...], preferred_element_type=_f32)
    xkj_ref[...] = jax.nn.silu(xkj) * g


def _pre_call(x, rbf, W_ji, b_ji, W_kj, b_kj, W_rbf):
    row = pl.BlockSpec((EBLK, F), lambda i: (i, 0))
    full = lambda shape: pl.BlockSpec(shape, lambda i, s=shape: tuple(0 for _ in s))
    return pl.pallas_call(
        _pre_body,
        grid=(NEB,),
        in_specs=[row,
                  pl.BlockSpec((EBLK, 6), lambda i: (i, 0)),
                  full((F, F)), full((1, F)), full((F, F)), full((1, F)),
                  full((6, F))],
        out_specs=[row, row],
        out_shape=[jax.ShapeDtypeStruct((E, F), _f32),
                   jax.ShapeDtypeStruct((E, F), _f32)],
    )(x, rbf, W_ji, b_ji, W_kj, b_kj, W_rbf)


GC = 64


@functools.cache
def _get_gather_sc():
    return pl.kernel(
        _gather_body,
        out_type=jax.ShapeDtypeStruct((TH, F), _f32),
        mesh=plsc.VectorSubcoreMesh(core_axis_name="c", subcore_axis_name="s",
                                    num_cores=NC, num_subcores=NS),
        scratch_types=[pltpu.VMEM((1, GC), jnp.int32),
                       pltpu.VMEM((1, GC), jnp.int32),
                       pltpu.VMEM((GC, F), _f32),
                       pltpu.VMEM((GC, F), _f32),
                       pltpu.SemaphoreType.DMA,
                       pltpu.SemaphoreType.DMA,
                       pltpu.SemaphoreType.DMA,
                       pltpu.SemaphoreType.DMA],
    )


def _gather_body(table_hbm, idx_hbm, out_hbm,
                 idx_a, idx_b, rows_a, rows_b, sga, sgb, soa, sob):
    wid = lax.axis_index("s") * NC + lax.axis_index("c")
    per_w = TH // (NC * NS)
    base = wid * per_w

    def body(kk, carry):
        o0 = base + kk * (2 * GC)
        o1 = o0 + GC
        pltpu.sync_copy(idx_hbm.at[pl.ds(o0, GC)], idx_a.at[0])
        ha = pltpu.async_copy(table_hbm.at[idx_a.at[0]], rows_a, sga)
        pltpu.sync_copy(idx_hbm.at[pl.ds(o1, GC)], idx_b.at[0])
        hb = pltpu.async_copy(table_hbm.at[idx_b.at[0]], rows_b, sgb)
        ha.wait()
        hoa = pltpu.async_copy(rows_a, out_hbm.at[pl.ds(o0, GC)], soa)
        hb.wait()
        hob = pltpu.async_copy(rows_b, out_hbm.at[pl.ds(o1, GC)], sob)
        hoa.wait()
        hob.wait()
        return carry

    lax.fori_loop(0, per_w // (2 * GC), body, 0)


def _make_bilin_body(half):
    def body(sbf_ref, g_ref, wsbf_ref, wf_ref, m_ref):
        pid = pl.program_id(0)
        sp = jnp.dot(sbf_ref[...], wsbf_ref[...], preferred_element_type=_f32)
        rows = ((half * (TH // TBLK) + pid) * TBLK
                + lax.broadcasted_iota(jnp.int32, (TBLK, 1), 0))
        sp = jnp.where(rows < T, sp, 0.0)
        gb = g_ref[...]
        acc = jnp.zeros((TBLK, F), _f32)
        for j in range(NB):
            acc += sp[:, j:j + 1] * jnp.dot(gb, wf_ref[j],
                                            preferred_element_type=_f32)
        m_ref[...] = acc
    return body


def _bilin_call(sbf, G, W_sbf, Wf, half):
    hb = TH // TBLK
    return pl.pallas_call(
        _make_bilin_body(half),
        grid=(hb,),
        in_specs=[pl.BlockSpec((TBLK, 42), lambda i: (half * hb + i, 0)),
                  pl.BlockSpec((TBLK, F), lambda i: (i, 0)),
                  pl.BlockSpec((42, NB), lambda i: (0, 0)),
                  pl.BlockSpec((NB, F, F), lambda i: (0, 0, 0))],
        out_specs=pl.BlockSpec((TBLK, F), lambda i: (i, 0)),
        out_shape=jax.ShapeDtypeStruct((TH, F), _f32),
    )(sbf, G, W_sbf, Wf)


EPS = ERNG // NS
_WCH = ((0, 128), (128, 128), (256, 128), (384, 96))


@functools.cache
def _get_scatter_sc():
    return pl.kernel(
        _scatter_body,
        out_type=jax.ShapeDtypeStruct((EPAD, F), _f32),
        mesh=plsc.VectorSubcoreMesh(core_axis_name="c", subcore_axis_name="s",
                                    num_cores=NC, num_subcores=NS),
        scratch_types=[pltpu.VMEM((TPAD // NS // CHUNK, CHUNK), jnp.int32),
                       pltpu.VMEM((1, CHUNK), jnp.int32),
                       pltpu.VMEM((1, CHUNK), jnp.int32),
                       pltpu.VMEM((CHUNK, F), _f32),
                       pltpu.VMEM((CHUNK, F), _f32),
                       pltpu.VMEM_SHARED((ERNG + TRASH, F), _f32),
                       pltpu.SemaphoreType.DMA,
                       pltpu.SemaphoreType.DMA,
                       pltpu.SemaphoreType.DMA,
                       pltpu.SemaphoreType.DMA],
    )


def _scatter_body(m0_hbm, m1_hbm, idx3_hbm, zeros_hbm, out_hbm,
                  idxbuf, idx_ta, idx_tb, mbuf_a, mbuf_b, acc,
                  sma, smb, saa, sab):
    c = lax.axis_index("c")
    s = lax.axis_index("s")
    per_s = TPAD // NS
    base = (s % (NS // 2)) * per_s
    nchunk = per_s // CHUNK

    pltpu.sync_copy(idx3_hbm.at[s], idxbuf)

    def transform(idx_t, k, e0):
        for j in range(CHUNK // 16):
            v = idxbuf[k, pl.ds(16 * j, 16)] - e0
            trash = ERNG + (16 * j) % TRASH + lax.iota(jnp.int32, 16)
            ok = (v >= 0) & (v < ERNG)
            idx_t[0, pl.ds(16 * j, 16)] = jnp.where(ok, v, trash)

    for p in range(NR // NC):
        r = NC * c + p
        e0 = r * ERNG

        pltpu.sync_copy(zeros_hbm, mbuf_a)
        for off, sz in _WCH:
            pltpu.sync_copy(mbuf_a.at[pl.ds(0, sz)],
                            acc.at[pl.ds(s * EPS + off, sz)])
        plsc.subcore_barrier()

        def run_loop(m_hbm):
            def body(kk, carry):
                k0 = 2 * kk
                o0 = base + k0 * CHUNK
                ha = pltpu.async_copy(m_hbm.at[pl.ds(o0, CHUNK)], mbuf_a, sma)
                hb = pltpu.async_copy(m_hbm.at[pl.ds(o0 + CHUNK, CHUNK)],
                                      mbuf_b, smb)
                transform(idx_ta, k0, e0)
                transform(idx_tb, k0 + 1, e0)
                ha.wait()
                haa = pltpu.async_copy(mbuf_a, acc.at[idx_ta.at[0]], saa,
                                       add=True)
                hb.wait()
                hab = pltpu.async_copy(mbuf_b, acc.at[idx_tb.at[0]], sab,
                                       add=True)
                haa.wait()
                hab.wait()
                return carry
            lax.fori_loop(0, per_s // (2 * CHUNK), body, 0)

        @pl.when(s < NS // 2)
        def _():
            run_loop(m0_hbm)

        @pl.when(s >= NS // 2)
        def _():
            run_loop(m1_hbm)

        plsc.subcore_barrier()

        for off, sz in _WCH:
            pltpu.sync_copy(acc.at[pl.ds(s * EPS + off, sz)],
                            mbuf_a.at[pl.ds(0, sz)])
            pltpu.sync_copy(mbuf_a.at[pl.ds(0, sz)],
                            out_hbm.at[pl.ds(e0 + s * EPS + off, sz)])


def _post_body(x_ref, xji_ref, red_ref,
               w1, b1, w2, b2, w3, b3, w4, b4, w5, b5, w6, b6, w7, b7,
               out_ref):
    act = jax.nn.silu

    def lin(v, w, b):
        return jnp.dot(v, w[...], preferred_element_type=_f32) + b[...]

    x2 = xji_ref[...] + red_ref[...]
    h = act(lin(x2, w1, b1))
    h = act(lin(h, w2, b2))
    x2 = x2 + h
    x2 = act(lin(x2, w3, b3))
    xo = x_ref[...] + x2
    h = act(lin(xo, w4, b4))
    h = act(lin(h, w5, b5))
    xo = xo + h
    h = act(lin(xo, w6, b6))
    h = act(lin(h, w7, b7))
    out_ref[...] = xo + h


def _post_call(x, x_ji, red, *wbs):
    row = pl.BlockSpec((EBLK, F), lambda i: (i, 0))
    wspec = pl.BlockSpec((F, F), lambda i: (0, 0))
    bspec = pl.BlockSpec((1, F), lambda i: (0, 0))
    return pl.pallas_call(
        _post_body,
        grid=(NEB,),
        in_specs=[row, row, row] + [wspec, bspec] * 7,
        out_specs=row,
        out_shape=jax.ShapeDtypeStruct((E, F), _f32),
    )(x, x_ji, red, *wbs)


def kernel(x, rbf, sbf, id_expand_kj, id_reduce_ji,
           W_rbf, W_sbf, W_ji, b_ji, W_kj, b_kj, W_bilin,
           W_bs0_0, b_bs0_0, W_bs0_1, b_bs0_1,
           W_fbs, b_fbs,
           W_as0_0, b_as0_0, W_as0_1, b_as0_1,
           W_as1_0, b_as1_0, W_as1_1, b_as1_1):
    b2 = lambda b: b.reshape(1, F)
    x_ji, x_kj = _pre_call(x, rbf, W_ji, b2(b_ji), W_kj, b2(b_kj), W_rbf)

    ide = jnp.pad(id_expand_kj.astype(jnp.int32), (0, TPAD - T))
    Wf = jnp.transpose(W_bilin, (1, 2, 0))

    G0 = _get_gather_sc()(x_kj, ide[:TH])
    m0 = _bilin_call(sbf, G0, W_sbf, Wf, 0)
    G1 = _get_gather_sc()(x_kj, ide[TH:])
    m1 = _bilin_call(sbf, G1, W_sbf, Wf, 1)

    idr = jnp.pad(id_reduce_ji.astype(jnp.int32), (0, TPAD - T))
    idr3 = idr.reshape(NS, TPAD // NS // CHUNK, CHUNK)
    zeros = jnp.zeros((CHUNK, F), _f32)
    red = _get_scatter_sc()(m0, m1, idr3, zeros)

    return _post_call(x, x_ji, red,
                      W_bs0_0, b2(b_bs0_0), W_bs0_1, b2(b_bs0_1),
                      W_fbs, b2(b_fbs),
                      W_as0_0, b2(b_as0_0), W_as0_1, b2(b_as0_1),
                      W_as1_0, b2(b_as1_0), W_as1_1, b2(b_as1_1))


# speedup vs baseline: 2.8611x; 1.0067x over previous
import functools

import jax
import jax.numpy as jnp
from jax import lax
from jax.experimental import pallas as pl
from jax.experimental.pallas import tpu as pltpu
from jax.experimental.pallas import tpu_sc as plsc

F = 128
NB = 8
E = 30000
T = 90000
TPAD = 90112
EBLK = 7680
TBLK = 4096
TH = TPAD // 2
NEB = (E + EBLK - 1) // EBLK
NTB = TPAD // TBLK
NC = 2
NS = 16
CHUNK = 128
NR = 4
EPAD = 30720
ERNG = EPAD // NR
TRASH = CHUNK

_f32 = jnp.float32


def _pre_body(x_ref, rbf_ref, wkj_ref, bkj_ref, wrbf_ref, xkj_ref):
    xb = x_ref[...]
    xkj = jnp.dot(xb, wkj_ref[...], preferred_element_type=_f32) + bkj_ref[...]
    g = jnp.dot(rbf_ref[...], wrbf_ref[...], preferred_element_type=_f32)
    xkj_ref[...] = jax.nn.silu(xkj) * g


def _pre_call(x, rbf, W_kj, b_kj, W_rbf):
    row = pl.BlockSpec((EBLK, F), lambda i: (i, 0))
    full = lambda shape: pl.BlockSpec(shape, lambda i, s=shape: tuple(0 for _ in s))
    return pl.pallas_call(
        _pre_body,
        grid=(NEB,),
        in_specs=[row,
                  pl.BlockSpec((EBLK, 6), lambda i: (i, 0)),
                  full((F, F)), full((1, F)),
                  full((6, F))],
        out_specs=row,
        out_shape=jax.ShapeDtypeStruct((E, F), _f32),
    )(x, rbf, W_kj, b_kj, W_rbf)


GC = 64


@functools.cache
def _get_gather_sc():
    return pl.kernel(
        _gather_body,
        out_type=jax.ShapeDtypeStruct((TH, F), _f32),
        mesh=plsc.VectorSubcoreMesh(core_axis_name="c", subcore_axis_name="s",
                                    num_cores=NC, num_subcores=NS),
        scratch_types=[pltpu.VMEM((1, GC), jnp.int32),
                       pltpu.VMEM((1, GC), jnp.int32),
                       pltpu.VMEM((GC, F), _f32),
                       pltpu.VMEM((GC, F), _f32),
                       pltpu.SemaphoreType.DMA,
                       pltpu.SemaphoreType.DMA,
                       pltpu.SemaphoreType.DMA,
                       pltpu.SemaphoreType.DMA],
    )


def _gather_body(table_hbm, idx_hbm, out_hbm,
                 idx_a, idx_b, rows_a, rows_b, sga, sgb, soa, sob):
    wid = lax.axis_index("s") * NC + lax.axis_index("c")
    per_w = TH // (NC * NS)
    base = wid * per_w

    def body(kk, carry):
        o0 = base + kk * (2 * GC)
        o1 = o0 + GC
        pltpu.sync_copy(idx_hbm.at[pl.ds(o0, GC)], idx_a.at[0])
        ha = pltpu.async_copy(table_hbm.at[idx_a.at[0]], rows_a, sga)
        pltpu.sync_copy(idx_hbm.at[pl.ds(o1, GC)], idx_b.at[0])
        hb = pltpu.async_copy(table_hbm.at[idx_b.at[0]], rows_b, sgb)
        ha.wait()
        hoa = pltpu.async_copy(rows_a, out_hbm.at[pl.ds(o0, GC)], soa)
        hb.wait()
        hob = pltpu.async_copy(rows_b, out_hbm.at[pl.ds(o1, GC)], sob)
        hoa.wait()
        hob.wait()
        return carry

    lax.fori_loop(0, per_w // (2 * GC), body, 0)


def _make_bilin_body(half):
    def body(sbf_ref, g_ref, wsbf_ref, wf_ref, m_ref):
        pid = pl.program_id(0)
        sp = jnp.dot(sbf_ref[...], wsbf_ref[...], preferred_element_type=_f32)
        rows = ((half * (TH // TBLK) + pid) * TBLK
                + lax.broadcasted_iota(jnp.int32, (TBLK, 1), 0))
        sp = jnp.where(rows < T, sp, 0.0)
        gb = g_ref[...]
        acc = jnp.zeros((TBLK, F), _f32)
        for j in range(NB):
            acc += sp[:, j:j + 1] * jnp.dot(gb, wf_ref[j],
                                            preferred_element_type=_f32)
        m_ref[...] = acc
    return body


def _bilin_call(sbf, G, W_sbf, Wf, half):
    hb = TH // TBLK
    return pl.pallas_call(
        _make_bilin_body(half),
        grid=(hb,),
        in_specs=[pl.BlockSpec((TBLK, 42), lambda i: (half * hb + i, 0)),
                  pl.BlockSpec((TBLK, F), lambda i: (i, 0)),
                  pl.BlockSpec((42, NB), lambda i: (0, 0)),
                  pl.BlockSpec((NB, F, F), lambda i: (0, 0, 0))],
        out_specs=pl.BlockSpec((TBLK, F), lambda i: (i, 0)),
        out_shape=jax.ShapeDtypeStruct((TH, F), _f32),
    )(sbf, G, W_sbf, Wf)


EPS = ERNG // NS
_WCH = ((0, 128), (128, 128), (256, 128), (384, 96))


@functools.cache
def _get_scatter_sc():
    return pl.kernel(
        _scatter_body,
        out_type=jax.ShapeDtypeStruct((EPAD, F), _f32),
        mesh=plsc.VectorSubcoreMesh(core_axis_name="c", subcore_axis_name="s",
                                    num_cores=NC, num_subcores=NS),
        scratch_types=[pltpu.VMEM((TPAD // NS // CHUNK, CHUNK), jnp.int32),
                       pltpu.VMEM((1, CHUNK), jnp.int32),
                       pltpu.VMEM((1, CHUNK), jnp.int32),
                       pltpu.VMEM((CHUNK, F), _f32),
                       pltpu.VMEM((CHUNK, F), _f32),
                       pltpu.VMEM_SHARED((ERNG + TRASH, F), _f32),
                       pltpu.SemaphoreType.DMA,
                       pltpu.SemaphoreType.DMA,
                       pltpu.SemaphoreType.DMA,
                       pltpu.SemaphoreType.DMA],
    )


def _scatter_body(m0_hbm, m1_hbm, idx3_hbm, zeros_hbm, out_hbm,
                  idxbuf, idx_ta, idx_tb, mbuf_a, mbuf_b, acc,
                  sma, smb, saa, sab):
    c = lax.axis_index("c")
    s = lax.axis_index("s")
    per_s = TPAD // NS
    base = (s % (NS // 2)) * per_s
    nchunk = per_s // CHUNK

    pltpu.sync_copy(idx3_hbm.at[s], idxbuf)

    def transform(idx_t, k, e0):
        for j in range(CHUNK // 16):
            v = idxbuf[k, pl.ds(16 * j, 16)] - e0
            trash = ERNG + (16 * j) % TRASH + lax.iota(jnp.int32, 16)
            ok = (v >= 0) & (v < ERNG)
            idx_t[0, pl.ds(16 * j, 16)] = jnp.where(ok, v, trash)

    for p in range(NR // NC):
        r = NC * c + p
        e0 = r * ERNG

        pltpu.sync_copy(zeros_hbm, mbuf_a)
        for off, sz in _WCH:
            pltpu.sync_copy(mbuf_a.at[pl.ds(0, sz)],
                            acc.at[pl.ds(s * EPS + off, sz)])
        plsc.subcore_barrier()

        def run_loop(m_hbm):
            def body(kk, carry):
                k0 = 2 * kk
                o0 = base + k0 * CHUNK
                ha = pltpu.async_copy(m_hbm.at[pl.ds(o0, CHUNK)], mbuf_a, sma)
                hb = pltpu.async_copy(m_hbm.at[pl.ds(o0 + CHUNK, CHUNK)],
                                      mbuf_b, smb)
                transform(idx_ta, k0, e0)
                transform(idx_tb, k0 + 1, e0)
                ha.wait()
                haa = pltpu.async_copy(mbuf_a, acc.at[idx_ta.at[0]], saa,
                                       add=True)
                hb.wait()
                hab = pltpu.async_copy(mbuf_b, acc.at[idx_tb.at[0]], sab,
                                       add=True)
                haa.wait()
                hab.wait()
                return carry
            lax.fori_loop(0, per_s // (2 * CHUNK), body, 0)

        @pl.when(s < NS // 2)
        def _():
            run_loop(m0_hbm)

        @pl.when(s >= NS // 2)
        def _():
            run_loop(m1_hbm)

        plsc.subcore_barrier()

        for off, sz in _WCH:
            pltpu.sync_copy(acc.at[pl.ds(s * EPS + off, sz)],
                            mbuf_a.at[pl.ds(0, sz)])
            pltpu.sync_copy(mbuf_a.at[pl.ds(0, sz)],
                            out_hbm.at[pl.ds(e0 + s * EPS + off, sz)])


def _post_body(x_ref, red_ref, wji, bji,
               w1, b1, w2, b2, w3, b3, w4, b4, w5, b5, w6, b6, w7, b7,
               out_ref):
    act = jax.nn.silu

    def lin(v, w, b):
        return jnp.dot(v, w[...], preferred_element_type=_f32) + b[...]

    x2 = act(lin(x_ref[...], wji, bji)) + red_ref[...]
    h = act(lin(x2, w1, b1))
    h = act(lin(h, w2, b2))
    x2 = x2 + h
    x2 = act(lin(x2, w3, b3))
    xo = x_ref[...] + x2
    h = act(lin(xo, w4, b4))
    h = act(lin(h, w5, b5))
    xo = xo + h
    h = act(lin(xo, w6, b6))
    h = act(lin(h, w7, b7))
    out_ref[...] = xo + h


def _post_call(x, red, *wbs):
    row = pl.BlockSpec((EBLK, F), lambda i: (i, 0))
    wspec = pl.BlockSpec((F, F), lambda i: (0, 0))
    bspec = pl.BlockSpec((1, F), lambda i: (0, 0))
    return pl.pallas_call(
        _post_body,
        grid=(NEB,),
        in_specs=[row, row] + [wspec, bspec] * 8,
        out_specs=row,
        out_shape=jax.ShapeDtypeStruct((E, F), _f32),
    )(x, red, *wbs)


def kernel(x, rbf, sbf, id_expand_kj, id_reduce_ji,
           W_rbf, W_sbf, W_ji, b_ji, W_kj, b_kj, W_bilin,
           W_bs0_0, b_bs0_0, W_bs0_1, b_bs0_1,
           W_fbs, b_fbs,
           W_as0_0, b_as0_0, W_as0_1, b_as0_1,
           W_as1_0, b_as1_0, W_as1_1, b_as1_1):
    b2 = lambda b: b.reshape(1, F)
    x_kj = _pre_call(x, rbf, W_kj, b2(b_kj), W_rbf)

    ide = jnp.pad(id_expand_kj.astype(jnp.int32), (0, TPAD - T))
    Wf = jnp.transpose(W_bilin, (1, 2, 0))

    G0 = _get_gather_sc()(x_kj, ide[:TH])
    m0 = _bilin_call(sbf, G0, W_sbf, Wf, 0)
    G1 = _get_gather_sc()(x_kj, ide[TH:])
    m1 = _bilin_call(sbf, G1, W_sbf, Wf, 1)

    idr = jnp.pad(id_reduce_ji.astype(jnp.int32), (0, TPAD - T))
    idr3 = idr.reshape(NS, TPAD // NS // CHUNK, CHUNK)
    zeros = jnp.zeros((CHUNK, F), _f32)
    red = _get_scatter_sc()(m0, m1, idr3, zeros)

    return _post_call(x, red,
                      W_ji, b2(b_ji),
                      W_bs0_0, b2(b_bs0_0), W_bs0_1, b2(b_bs0_1),
                      W_fbs, b2(b_fbs),
                      W_as0_0, b2(b_as0_0), W_as0_1, b2(b_as0_1),
                      W_as1_0, b2(b_as1_0), W_as1_1, b2(b_as1_1))
